# baseline (speedup 1.0000x reference)
import functools

import jax
import jax.numpy as jnp
from jax import lax
from jax.experimental import pallas as pl
from jax.experimental.pallas import tpu as pltpu
from jax.experimental.pallas import tpu_sc as plsc

B, T, C, D = 16, 128, 16, 2048
L = 21
N = 4
TP = T - (N - 1)
BT = B * T
DC = 1024

LPAD = 32
_NW = 32
_TW = T // _NW


def _sc_hist_body(inp_hbm, out_hbm, inp_v, cnt_v):
    ci = lax.axis_index("c")
    si = lax.axis_index("s")
    wid = si * 2 + ci
    t0 = wid * _TW
    pltpu.sync_copy(inp_hbm.at[:, pl.ds(t0, _TW), :], inp_v)
    lanes_lo = lax.iota(jnp.int32, 16)
    lanes_hi = lanes_lo + 16
    zero = jnp.zeros((B,), jnp.float32)

    def tstep(tt, carry):
        for b in range(B):
            x = inp_v[b, tt, :]
            lev = (x - 0.0) / 20.0 * 20.0
            t = lev.astype(jnp.int32)
            f = lev - t.astype(jnp.float32)
            up = jnp.where(f > 0.5, 1, jnp.where(f == 0.5, t & 1, 0))
            idx = jnp.clip(t + up, 0, L - 1)
            lo = zero
            hi = zero
            for c in range(C):
                s = idx[c]
                lo = lo + jnp.where(lanes_lo == s, 1.0, 0.0)
                hi = hi + jnp.where(lanes_hi == s, 1.0, 0.0)
            r = tt * B + b
            cnt_v[r, pl.ds(0, 16)] = lo
            cnt_v[r, pl.ds(16, 16)] = hi
        return carry

    lax.fori_loop(0, _TW, tstep, 0)
    pltpu.sync_copy(cnt_v, out_hbm.at[pl.ds(t0 * B, _TW * B), :])


def _sc_hist(inp):
    run = functools.partial(
        pl.kernel,
        mesh=plsc.VectorSubcoreMesh(core_axis_name="c", subcore_axis_name="s"),
        out_type=jax.ShapeDtypeStruct((BT, LPAD), jnp.float32),
        scratch_types=[
            pltpu.VMEM((B, _TW, C), jnp.float32),
            pltpu.VMEM((_TW * B, LPAD), jnp.float32),
        ],
    )(_sc_hist_body)
    return run(inp)


def _tc_body(cnt_ref, sw_ref, tw_ref, pm_ref, out_ref):
    counts = cnt_ref[...][:, :L]
    s = jnp.dot(counts, sw_ref[...],
                preferred_element_type=jnp.float32)
    tw = tw_ref[...]
    twf = jnp.broadcast_to(tw[:, None, :], (T, B, DC)).reshape(BT, DC)
    samples = s * twf
    g = (samples[0:TP * B]
         * samples[B:(TP + 1) * B]
         * samples[2 * B:(TP + 2) * B]
         * samples[3 * B:(TP + 3) * B])
    acc = jnp.sum(g.reshape(TP, B, DC), axis=0)
    p = pm_ref[0, :] * pm_ref[1, :] * pm_ref[2, :]
    v = acc * p[None, :]
    out_ref[...] = jnp.where(v > 0, 1.0, -1.0)


def kernel(input, signals_weight, channels_weight, timestamps_weight, permute_hv):
    del channels_weight
    counts = _sc_hist(input)
    return pl.pallas_call(
        _tc_body,
        grid=(D // DC,),
        in_specs=[
            pl.BlockSpec((BT, LPAD), lambda d: (0, 0)),
            pl.BlockSpec((L, DC), lambda d: (0, d)),
            pl.BlockSpec((T, DC), lambda d: (0, d)),
            pl.BlockSpec((N - 1, DC), lambda d: (0, d)),
        ],
        out_specs=pl.BlockSpec((B, DC), lambda d: (0, d)),
        out_shape=jax.ShapeDtypeStruct((B, D), jnp.float32),
    )(counts, signals_weight, timestamps_weight, permute_hv)


# speedup vs baseline: 1.0752x; 1.0752x over previous
import functools

import jax
import jax.numpy as jnp
from jax import lax
from jax.experimental import pallas as pl
from jax.experimental.pallas import tpu as pltpu
from jax.experimental.pallas import tpu_sc as plsc

B, T, C, D = 16, 128, 16, 2048
L = 21
N = 4
TP = T - (N - 1)
BT = B * T
DC = 1024

_NW = 16
_TW = T // _NW


def _sc_hist_body(inp_hbm, out_hbm, inp_v, cnt_v):
    ci = lax.axis_index("c")
    si = lax.axis_index("s")
    t0 = (ci * 8 + si) * _TW

    @pl.when(si < 8)
    def _():
        pltpu.sync_copy(inp_hbm.at[pl.ds(t0 * C * B, _TW * C * B)], inp_v)
        zero = jnp.zeros((B,), jnp.float32)

        def tstep(tt, carry):
            accs = [zero] * L
            for c in range(C):
                x = inp_v[pl.ds(tt * C * B + c * B, B)]
                lev = (x - 0.0) / 20.0 * 20.0
                t = lev.astype(jnp.int32)
                f = lev - t.astype(jnp.float32)
                up = jnp.where(f > 0.5, 1, jnp.where(f == 0.5, t & 1, 0))
                idx = jnp.clip(t + up, 0, L - 1)
                for l in range(L):
                    accs[l] = accs[l] + jnp.where(idx == l, 1.0, 0.0)
            for l in range(L):
                cnt_v[l, pl.ds(tt * B, B)] = accs[l]
            return carry

        lax.fori_loop(0, _TW, tstep, 0)
        pltpu.sync_copy(cnt_v, out_hbm.at[:, pl.ds(t0 * B, _TW * B)])


def _sc_hist(inp):
    run = functools.partial(
        pl.kernel,
        mesh=plsc.VectorSubcoreMesh(core_axis_name="c", subcore_axis_name="s"),
        out_type=jax.ShapeDtypeStruct((L, BT), jnp.float32),
        scratch_types=[
            pltpu.VMEM((_TW * C * B,), jnp.float32),
            pltpu.VMEM((L, _TW * B), jnp.float32),
        ],
    )(_sc_hist_body)
    return run(inp)


def _tc_body(cnt_ref, sw_ref, tw_ref, pm_ref, out_ref):
    counts_t = cnt_ref[...]
    s = lax.dot_general(counts_t, sw_ref[...],
                        (((0,), (0,)), ((), ())),
                        preferred_element_type=jnp.float32)
    tw = tw_ref[...]
    twf = jnp.broadcast_to(tw[:, None, :], (T, B, DC)).reshape(BT, DC)
    samples = s * twf
    g = (samples[0:TP * B]
         * samples[B:(TP + 1) * B]
         * samples[2 * B:(TP + 2) * B]
         * samples[3 * B:(TP + 3) * B])
    acc = jnp.sum(g.reshape(TP, B, DC), axis=0)
    p = pm_ref[0, :] * pm_ref[1, :] * pm_ref[2, :]
    v = acc * p[None, :]
    out_ref[...] = jnp.where(v > 0, 1.0, -1.0)


def kernel(input, signals_weight, channels_weight, timestamps_weight, permute_hv):
    del channels_weight
    inp3 = input.reshape(B, T * C).T.reshape(T * C * B)
    counts_t = _sc_hist(inp3)
    return pl.pallas_call(
        _tc_body,
        grid=(D // DC,),
        in_specs=[
            pl.BlockSpec((L, BT), lambda d: (0, 0)),
            pl.BlockSpec((L, DC), lambda d: (0, d)),
            pl.BlockSpec((T, DC), lambda d: (0, d)),
            pl.BlockSpec((N - 1, DC), lambda d: (0, d)),
        ],
        out_specs=pl.BlockSpec((B, DC), lambda d: (0, d)),
        out_shape=jax.ShapeDtypeStruct((B, D), jnp.float32),
    )(counts_t, signals_weight, timestamps_weight, permute_hv)


# speedup vs baseline: 1.0844x; 1.0085x over previous
import functools

import jax
import jax.numpy as jnp
from jax import lax
from jax.experimental import pallas as pl
from jax.experimental.pallas import tpu as pltpu
from jax.experimental.pallas import tpu_sc as plsc

B, T, C, D = 16, 128, 16, 2048
L = 21
N = 4
TP = T - (N - 1)
BT = B * T
DC = 2048

_NW = 16
_TW = T // _NW


def _sc_hist_body(inp_hbm, out_hbm, inp_v, cnt_v):
    ci = lax.axis_index("c")
    si = lax.axis_index("s")
    t0 = (ci * 8 + si) * _TW

    @pl.when(si < 8)
    def _():
        pltpu.sync_copy(inp_hbm.at[pl.ds(t0 * C * B, _TW * C * B)], inp_v)
        zero = jnp.zeros((B,), jnp.float32)

        def tstep(tt, carry):
            accs = [zero] * L
            for c in range(C):
                x = inp_v[pl.ds(tt * C * B + c * B, B)]
                lev = (x - 0.0) / 20.0 * 20.0
                t = lev.astype(jnp.int32)
                f = lev - t.astype(jnp.float32)
                up = jnp.where(f > 0.5, 1, jnp.where(f == 0.5, t & 1, 0))
                idx = jnp.clip(t + up, 0, L - 1)
                for l in range(L):
                    accs[l] = accs[l] + jnp.where(idx == l, 1.0, 0.0)
            for l in range(L):
                cnt_v[l, pl.ds(tt * B, B)] = accs[l]
            return carry

        lax.fori_loop(0, _TW, tstep, 0)
        pltpu.sync_copy(cnt_v, out_hbm.at[:, pl.ds(t0 * B, _TW * B)])


def _sc_hist(inp):
    run = functools.partial(
        pl.kernel,
        mesh=plsc.VectorSubcoreMesh(core_axis_name="c", subcore_axis_name="s"),
        out_type=jax.ShapeDtypeStruct((L, BT), jnp.float32),
        scratch_types=[
            pltpu.VMEM((_TW * C * B,), jnp.float32),
            pltpu.VMEM((L, _TW * B), jnp.float32),
        ],
    )(_sc_hist_body)
    return run(inp)


def _tc_body(cnt_ref, sw_ref, tw_ref, pm_ref, out_ref):
    counts_t = cnt_ref[...]
    s = lax.dot_general(counts_t, sw_ref[...],
                        (((0,), (0,)), ((), ())),
                        preferred_element_type=jnp.float32)
    tw = tw_ref[...]
    twf = jnp.broadcast_to(tw[:, None, :], (T, B, DC)).reshape(BT, DC)
    samples = s * twf
    g = (samples[0:TP * B]
         * samples[B:(TP + 1) * B]
         * samples[2 * B:(TP + 2) * B]
         * samples[3 * B:(TP + 3) * B])
    acc = jnp.sum(g.reshape(TP, B, DC), axis=0)
    p = pm_ref[0, :] * pm_ref[1, :] * pm_ref[2, :]
    v = acc * p[None, :]
    out_ref[...] = jnp.where(v > 0, 1.0, -1.0)


def kernel(input, signals_weight, channels_weight, timestamps_weight, permute_hv):
    del channels_weight
    inp3 = input.reshape(B, T * C).T.reshape(T * C * B)
    counts_t = _sc_hist(inp3)
    return pl.pallas_call(
        _tc_body,
        grid=(D // DC,),
        in_specs=[
            pl.BlockSpec((L, BT), lambda d: (0, 0)),
            pl.BlockSpec((L, DC), lambda d: (0, d)),
            pl.BlockSpec((T, DC), lambda d: (0, d)),
            pl.BlockSpec((N - 1, DC), lambda d: (0, d)),
        ],
        out_specs=pl.BlockSpec((B, DC), lambda d: (0, d)),
        out_shape=jax.ShapeDtypeStruct((B, D), jnp.float32),
    )(counts_t, signals_weight, timestamps_weight, permute_hv)
